# baseline (speedup 1.0000x reference)
import jax
import jax.numpy as jnp
from jax import lax
from jax.experimental import pallas as pl
from jax.experimental.pallas import tpu as pltpu
from jax.experimental.pallas import tpu_sc as plsc

_NC = 2
_NS = 16
_NW = _NC * _NS
_L = 16

_N = 10000
_E = 320000
_D = 128
_DH = _D // 2

_N_PAD = 10240
_TRASH = _N
_EPT = _E // _NW

_CWA = 128
_CREG = 81
_RSZ = _CREG * _CWA

_RPT = _N_PAD // _NS
_BR = 400


def _deg_body(fei_hbm, clus_hbm, hist_out, csrc_out, cdst_out, cnt_out,
              src_v, dst_v, clus_v, hist_v, csrc_v, cdst_v, cnt_v):
    c = lax.axis_index("c")
    s = lax.axis_index("s")
    wid = s * _NC + c
    pltpu.sync_copy(clus_hbm, clus_v)
    pltpu.sync_copy(fei_hbm.at[0, pl.ds(wid * _EPT, _EPT)], src_v)
    pltpu.sync_copy(fei_hbm.at[1, pl.ds(wid * _EPT, _EPT)], dst_v)

    zeros16 = jnp.zeros((_L,), jnp.float32)

    @pl.loop(0, _N // _L)
    def _zero(i):
        hist_v[pl.ds(i * _L, _L)] = zeros16

    zeros16i = jnp.zeros((_L,), jnp.int32)
    trash16 = jnp.full((_L,), _TRASH, jnp.int32)

    @pl.loop(0, _RSZ // _L)
    def _pre(i):
        csrc_v[pl.ds(i * _L, _L)] = zeros16i
        cdst_v[pl.ds(i * _L, _L)] = trash16

    ones16 = jnp.ones((_L,), jnp.float32)

    @pl.loop(0, _EPT // _L, init_carry=jnp.int32(0))
    def _group(g, off):
        sl = pl.ds(g * _L, _L)
        sidx = src_v[sl]
        didx = dst_v[sl]
        cs = plsc.load_gather(clus_v, [sidx])
        cd = plsc.load_gather(clus_v, [didx])
        m = cs == cd
        plsc.addupdate_scatter(hist_v, [didx], ones16, mask=m)
        plsc.store_compressed(csrc_v.at[pl.ds(off, _L)], sidx, mask=m)
        plsc.store_compressed(cdst_v.at[pl.ds(off, _L)], didx, mask=m)
        return off + plsc.all_reduce_population_count(m)[0]

    off = _group
    nch = (off + _CWA - 1) // _CWA
    cnt_v[...] = jnp.full((_L,), nch, jnp.int32)

    pltpu.sync_copy(hist_v, hist_out.at[wid])
    pltpu.sync_copy(csrc_v.at[pl.ds(0, _RSZ)], csrc_out.at[wid])
    pltpu.sync_copy(cdst_v.at[pl.ds(0, _RSZ)], cdst_out.at[wid])
    pltpu.sync_copy(cnt_v, cnt_out.at[wid])


def _agg_body(y_hbm, csrc_hbm, cdst_hbm, cnt_hbm, agg_out,
              src_v, dst_v, cnt_v, rows0, rows1, rows2, rows3, zero_v, agg_sh,
              gsem0, gsem1, gsem2, gsem3, ssem0, ssem1, ssem2, ssem3):
    rows = (rows0, rows1, rows2, rows3)
    gsem = (gsem0, gsem1, gsem2, gsem3)
    ssem = (ssem0, ssem1, ssem2, ssem3)
    c = lax.axis_index("c")
    s = lax.axis_index("s")

    zeros16 = jnp.zeros((_L,), jnp.float32)

    @pl.loop(0, 16)
    def _zbuf(i):
        for k in range(_DH // _L):
            zero_v[i, pl.ds(k * _L, _L)] = zeros16

    @pl.loop(0, _RPT // 16)
    def _zstripe(i):
        pltpu.sync_copy(zero_v, agg_sh.at[pl.ds(s * _RPT + i * 16, 16)])

    ytab = y_hbm.at[c]
    pltpu.sync_copy(cnt_hbm.at[s], cnt_v)
    pltpu.sync_copy(csrc_hbm.at[2 * s], src_v.at[0])
    pltpu.sync_copy(csrc_hbm.at[2 * s + 1], src_v.at[1])
    pltpu.sync_copy(cdst_hbm.at[2 * s], dst_v.at[0])
    pltpu.sync_copy(cdst_hbm.at[2 * s + 1], dst_v.at[1])
    n0 = cnt_v[0][0]
    n1 = cnt_v[1][0]
    tot = n0 + n1
    nq = (jnp.maximum(tot, 1) + 3) // 4
    last4 = 4 * nq

    def chref(arr, j):
        in0 = j < n0
        inr = j < tot
        r_sel = jnp.where(in0 | (~inr), 0, 1)
        ch = jnp.where(in0, j, jnp.where(inr, j - n0, _CREG - 1))
        return arr.at[r_sel, ch]

    plsc.subcore_barrier()
    for k in range(4):
        pltpu.async_copy(ytab.at[chref(src_v, k)], rows[k], gsem[k])

    @pl.loop(0, nq)
    def _pipe(i):
        base = 4 * i
        for k in range(4):
            j = base + k
            pltpu.make_async_copy(ytab.at[chref(src_v, j)], rows[k], gsem[k]).wait()
            pltpu.async_copy(rows[k], agg_sh.at[chref(dst_v, j)], ssem[k], add=True)
        for k in range(4):
            j = base + k

            @pl.when(j + 4 < last4)
            def _refill(j=j, k=k):
                pltpu.make_async_copy(
                    rows[k], agg_sh.at[chref(dst_v, j)], ssem[k]).wait()
                pltpu.async_copy(ytab.at[chref(src_v, j + 4)], rows[k], gsem[k])

    for k in range(4):
        jl = last4 - 4 + k
        pltpu.make_async_copy(rows[k], agg_sh.at[chref(dst_v, jl)], ssem[k]).wait()

    plsc.subcore_barrier()
    pltpu.sync_copy(agg_sh.at[pl.ds(s * _RPT, _RPT)],
                    agg_out.at[c, pl.ds(s * _RPT, _RPT)])


def _dy_body(hist_ref, x_ref, w_ref, y_ref, dinv_ref):
    deg = jnp.sum(hist_ref[...], axis=0) + 1.0
    dinv = lax.rsqrt(deg)[:, None]
    dinv_ref[...] = dinv
    xw = jnp.dot(x_ref[...], w_ref[...], preferred_element_type=jnp.float32)
    y = xw * dinv
    y_ref[0] = y[:, :_DH]
    y_ref[1] = y[:, _DH:]


def _combine_body(agg_ref, y_ref, dinv_ref, b_ref, o_ref):
    tot = jnp.concatenate([agg_ref[0] + y_ref[0], agg_ref[1] + y_ref[1]], axis=1)
    o_ref[...] = dinv_ref[...] * tot + b_ref[...]


def _sc_mesh():
    return plsc.VectorSubcoreMesh(core_axis_name="c", subcore_axis_name="s")


def _deg_call(fei, clus):
    f = pl.kernel(
        _deg_body,
        out_type=(
            jax.ShapeDtypeStruct((_NW, _N), jnp.float32),
            jax.ShapeDtypeStruct((_NW, _RSZ), jnp.int32),
            jax.ShapeDtypeStruct((_NW, _RSZ), jnp.int32),
            jax.ShapeDtypeStruct((_NW, _L), jnp.int32),
        ),
        mesh=_sc_mesh(),
        scratch_types=[
            pltpu.VMEM((_EPT,), jnp.int32),
            pltpu.VMEM((_EPT,), jnp.int32),
            pltpu.VMEM((_N,), jnp.int32),
            pltpu.VMEM((_N,), jnp.float32),
            pltpu.VMEM((_RSZ + _L,), jnp.int32),
            pltpu.VMEM((_RSZ + _L,), jnp.int32),
            pltpu.VMEM((_L,), jnp.int32),
        ],
        compiler_params=pltpu.CompilerParams(
            needs_layout_passes=False, use_tc_tiling_on_sc=False),
    )
    return f(fei, clus)


def _agg_call(y2, csrc_a, cdst_a, cnt):
    f = pl.kernel(
        _agg_body,
        out_type=jax.ShapeDtypeStruct((_NC, _N_PAD, _DH), jnp.float32),
        mesh=_sc_mesh(),
        scratch_types=[
            pltpu.VMEM((2, _CREG, _CWA), jnp.int32),
            pltpu.VMEM((2, _CREG, _CWA), jnp.int32),
            pltpu.VMEM((2, _L), jnp.int32),
            pltpu.VMEM((_CWA, _DH), jnp.float32),
            pltpu.VMEM((_CWA, _DH), jnp.float32),
            pltpu.VMEM((_CWA, _DH), jnp.float32),
            pltpu.VMEM((_CWA, _DH), jnp.float32),
            pltpu.VMEM((16, _DH), jnp.float32),
            pltpu.VMEM_SHARED((_N_PAD, _DH), jnp.float32),
            pltpu.SemaphoreType.DMA,
            pltpu.SemaphoreType.DMA,
            pltpu.SemaphoreType.DMA,
            pltpu.SemaphoreType.DMA,
            pltpu.SemaphoreType.DMA,
            pltpu.SemaphoreType.DMA,
            pltpu.SemaphoreType.DMA,
            pltpu.SemaphoreType.DMA,
        ],
        compiler_params=pltpu.CompilerParams(
            needs_layout_passes=False, use_tc_tiling_on_sc=False),
    )
    return f(y2, csrc_a, cdst_a, cnt)


def _dy_call(hist, x, w):
    return pl.pallas_call(
        _dy_body,
        out_shape=(
            jax.ShapeDtypeStruct((_NC, _N, _DH), jnp.float32),
            jax.ShapeDtypeStruct((_N, 1), jnp.float32),
        ),
    )(hist, x, w)


def _combine_call(agg, y2, dinv, b2):
    return pl.pallas_call(
        _combine_body,
        grid=(_N // _BR,),
        in_specs=[
            pl.BlockSpec((_NC, _BR, _DH), lambda i: (0, i, 0)),
            pl.BlockSpec((_NC, _BR, _DH), lambda i: (0, i, 0)),
            pl.BlockSpec((_BR, 1), lambda i: (i, 0)),
            pl.BlockSpec((1, _D), lambda i: (0, 0)),
        ],
        out_specs=pl.BlockSpec((_BR, _D), lambda i: (i, 0)),
        out_shape=jax.ShapeDtypeStruct((_N, _D), jnp.float32),
    )(agg, y2, dinv, b2)


def kernel(X, W, b, cluster_assignment, full_edge_index):
    n, d = X.shape
    hist, csrc, cdst, cnt = _deg_call(full_edge_index, cluster_assignment)
    y2, dinv = _dy_call(hist, X, W)
    agg = _agg_call(y2,
                    csrc.reshape(_NW, _CREG, _CWA),
                    cdst.reshape(_NW, _CREG, _CWA),
                    cnt.reshape(_NS, 2, _L))
    return _combine_call(agg, y2, dinv, b.reshape(1, d))


# speedup vs baseline: 1.2060x; 1.2060x over previous
import jax
import jax.numpy as jnp
from jax import lax
from jax.experimental import pallas as pl
from jax.experimental.pallas import tpu as pltpu
from jax.experimental.pallas import tpu_sc as plsc

_NC = 2
_NS = 16
_NW = _NC * _NS
_L = 16

_N = 10000
_E = 320000
_D = 128
_DH = _D // 2

_N_PAD = 10240
_TRASH = _N
_EPT = _E // _NW

_CWA = 128
_CREG = 81
_RSZ = _CREG * _CWA

_RPT = _N_PAD // _NS
_BR = 400


def _deg_body(fei_hbm, clus_hbm, hist_out, csrc_out, cdst_out, cnt_out,
              src_v, dst_v, clus_v, hist_v, csrc_v, cdst_v, cnt_v):
    c = lax.axis_index("c")
    s = lax.axis_index("s")
    wid = s * _NC + c
    pltpu.sync_copy(clus_hbm, clus_v)
    pltpu.sync_copy(fei_hbm.at[0, pl.ds(wid * _EPT, _EPT)], src_v)
    pltpu.sync_copy(fei_hbm.at[1, pl.ds(wid * _EPT, _EPT)], dst_v)

    zeros16 = jnp.zeros((_L,), jnp.float32)

    @pl.loop(0, _N // _L)
    def _zero(i):
        hist_v[pl.ds(i * _L, _L)] = zeros16

    zeros16i = jnp.zeros((_L,), jnp.int32)
    trash16 = jnp.full((_L,), _TRASH, jnp.int32)

    @pl.loop(0, _RSZ // _L)
    def _pre(i):
        csrc_v[pl.ds(i * _L, _L)] = zeros16i
        cdst_v[pl.ds(i * _L, _L)] = trash16

    ones16 = jnp.ones((_L,), jnp.float32)

    @pl.loop(0, _EPT // _L, init_carry=jnp.int32(0))
    def _group(g, off):
        sl = pl.ds(g * _L, _L)
        sidx = src_v[sl]
        didx = dst_v[sl]
        cs = plsc.load_gather(clus_v, [sidx])
        cd = plsc.load_gather(clus_v, [didx])
        m = cs == cd
        plsc.addupdate_scatter(hist_v, [didx], ones16, mask=m)
        plsc.store_compressed(csrc_v.at[pl.ds(off, _L)], sidx, mask=m)
        plsc.store_compressed(cdst_v.at[pl.ds(off, _L)], didx, mask=m)
        return off + plsc.all_reduce_population_count(m)[0]

    off = _group
    nch = (off + _CWA - 1) // _CWA
    cnt_v[...] = jnp.full((_L,), nch, jnp.int32)

    pltpu.sync_copy(hist_v, hist_out.at[wid])
    pltpu.sync_copy(csrc_v.at[pl.ds(0, _RSZ)], csrc_out.at[wid])
    pltpu.sync_copy(cdst_v.at[pl.ds(0, _RSZ)], cdst_out.at[wid])
    pltpu.sync_copy(cnt_v, cnt_out.at[wid])


def _agg_body(y_hbm, csrc_hbm, cdst_hbm, cnt_hbm, agg_out,
              src_v, dst_v, cnt_v, rows0, rows1, zero_v, agg_sh, sem0, sem1):

    c = lax.axis_index("c")
    s = lax.axis_index("s")

    zeros16 = jnp.zeros((_L,), jnp.float32)

    @pl.loop(0, 16)
    def _zbuf(i):
        for k in range(_DH // _L):
            zero_v[i, pl.ds(k * _L, _L)] = zeros16

    @pl.loop(0, _RPT // 16)
    def _zstripe(i):
        pltpu.sync_copy(zero_v, agg_sh.at[pl.ds(s * _RPT + i * 16, 16)])

    ytab = y_hbm.at[c]
    pltpu.sync_copy(cnt_hbm.at[s], cnt_v)
    pltpu.sync_copy(csrc_hbm.at[2 * s], src_v.at[0])
    pltpu.sync_copy(csrc_hbm.at[2 * s + 1], src_v.at[1])
    pltpu.sync_copy(cdst_hbm.at[2 * s], dst_v.at[0])
    pltpu.sync_copy(cdst_hbm.at[2 * s + 1], dst_v.at[1])
    n0 = cnt_v[0][0]
    n1 = cnt_v[1][0]
    tot = n0 + n1
    npair = (jnp.maximum(tot, 1) + 1) // 2
    last = 2 * npair

    def chref(arr, j):
        in0 = j < n0
        inr = j < tot
        r_sel = jnp.where(in0 | (~inr), 0, 1)
        ch = jnp.where(in0, j, jnp.where(inr, j - n0, _CREG - 1))
        return arr.at[r_sel, ch]

    plsc.subcore_barrier()
    pltpu.async_copy(ytab.at[chref(src_v, 0)], rows0, sem0)

    @pl.loop(0, npair)
    def _pipe(i):
        j0 = 2 * i
        j1 = j0 + 1
        pltpu.async_copy(ytab.at[chref(src_v, j1)], rows1, sem1)
        pltpu.make_async_copy(ytab.at[chref(src_v, j0)], rows0, sem0).wait()
        pltpu.sync_copy(rows0, agg_sh.at[chref(dst_v, j0)], add=True)

        @pl.when(j1 + 1 < last)
        def _start_next():
            pltpu.async_copy(ytab.at[chref(src_v, j1 + 1)], rows0, sem0)

        pltpu.make_async_copy(ytab.at[chref(src_v, j1)], rows1, sem1).wait()
        pltpu.sync_copy(rows1, agg_sh.at[chref(dst_v, j1)], add=True)

    plsc.subcore_barrier()
    pltpu.sync_copy(agg_sh.at[pl.ds(s * _RPT, _RPT)],
                    agg_out.at[c, pl.ds(s * _RPT, _RPT)])


def _dy_body(hist_ref, x_ref, w_ref, y_ref, dinv_ref):
    deg = jnp.sum(hist_ref[...], axis=0) + 1.0
    dinv = lax.rsqrt(deg)[:, None]
    dinv_ref[...] = dinv
    xw = jnp.dot(x_ref[...], w_ref[...], preferred_element_type=jnp.float32)
    y = xw * dinv
    y_ref[0] = y[:, :_DH]
    y_ref[1] = y[:, _DH:]


def _combine_body(agg_ref, y_ref, dinv_ref, b_ref, o_ref):
    tot = jnp.concatenate([agg_ref[0] + y_ref[0], agg_ref[1] + y_ref[1]], axis=1)
    o_ref[...] = dinv_ref[...] * tot + b_ref[...]


def _sc_mesh():
    return plsc.VectorSubcoreMesh(core_axis_name="c", subcore_axis_name="s")


def _deg_call(fei, clus):
    f = pl.kernel(
        _deg_body,
        out_type=(
            jax.ShapeDtypeStruct((_NW, _N), jnp.float32),
            jax.ShapeDtypeStruct((_NW, _RSZ), jnp.int32),
            jax.ShapeDtypeStruct((_NW, _RSZ), jnp.int32),
            jax.ShapeDtypeStruct((_NW, _L), jnp.int32),
        ),
        mesh=_sc_mesh(),
        scratch_types=[
            pltpu.VMEM((_EPT,), jnp.int32),
            pltpu.VMEM((_EPT,), jnp.int32),
            pltpu.VMEM((_N,), jnp.int32),
            pltpu.VMEM((_N,), jnp.float32),
            pltpu.VMEM((_RSZ + _L,), jnp.int32),
            pltpu.VMEM((_RSZ + _L,), jnp.int32),
            pltpu.VMEM((_L,), jnp.int32),
        ],
        compiler_params=pltpu.CompilerParams(
            needs_layout_passes=False, use_tc_tiling_on_sc=False),
    )
    return f(fei, clus)


def _agg_call(y2, csrc_a, cdst_a, cnt):
    f = pl.kernel(
        _agg_body,
        out_type=jax.ShapeDtypeStruct((_NC, _N_PAD, _DH), jnp.float32),
        mesh=_sc_mesh(),
        scratch_types=[
            pltpu.VMEM((2, _CREG, _CWA), jnp.int32),
            pltpu.VMEM((2, _CREG, _CWA), jnp.int32),
            pltpu.VMEM((2, _L), jnp.int32),
            pltpu.VMEM((_CWA, _DH), jnp.float32),
            pltpu.VMEM((_CWA, _DH), jnp.float32),
            pltpu.VMEM((16, _DH), jnp.float32),
            pltpu.VMEM_SHARED((_N_PAD, _DH), jnp.float32),
            pltpu.SemaphoreType.DMA,
            pltpu.SemaphoreType.DMA,
        ],
        compiler_params=pltpu.CompilerParams(
            needs_layout_passes=False, use_tc_tiling_on_sc=False),
    )
    return f(y2, csrc_a, cdst_a, cnt)


def _dy_call(hist, x, w):
    return pl.pallas_call(
        _dy_body,
        out_shape=(
            jax.ShapeDtypeStruct((_NC, _N, _DH), jnp.float32),
            jax.ShapeDtypeStruct((_N, 1), jnp.float32),
        ),
    )(hist, x, w)


def _combine_call(agg, y2, dinv, b2):
    return pl.pallas_call(
        _combine_body,
        grid=(_N // _BR,),
        in_specs=[
            pl.BlockSpec((_NC, _BR, _DH), lambda i: (0, i, 0)),
            pl.BlockSpec((_NC, _BR, _DH), lambda i: (0, i, 0)),
            pl.BlockSpec((_BR, 1), lambda i: (i, 0)),
            pl.BlockSpec((1, _D), lambda i: (0, 0)),
        ],
        out_specs=pl.BlockSpec((_BR, _D), lambda i: (i, 0)),
        out_shape=jax.ShapeDtypeStruct((_N, _D), jnp.float32),
    )(agg, y2, dinv, b2)


def kernel(X, W, b, cluster_assignment, full_edge_index):
    n, d = X.shape
    hist, csrc, cdst, cnt = _deg_call(full_edge_index, cluster_assignment)
    y2, dinv = _dy_call(hist, X, W)
    agg = _agg_call(y2,
                    csrc.reshape(_NW, _CREG, _CWA),
                    cdst.reshape(_NW, _CREG, _CWA),
                    cnt.reshape(_NS, 2, _L))
    return _combine_call(agg, y2, dinv, b.reshape(1, d))
